# baseline (speedup 1.0000x reference)
import functools
import math

import jax
import jax.numpy as jnp
from jax import lax
from jax.experimental import pallas as pl
from jax.experimental.pallas import tpu as pltpu
from jax.experimental.pallas import tpu_sc as plsc

D_MODEL = 1024
SCALE = math.sqrt(D_MODEL)

_info = plsc.get_sparse_core_info()
_NC = _info.num_cores
_NS = _info.num_subcores
_LANES = _info.num_lanes
_NW = _NC * _NS

_NG = 8
_NSB = 4


@functools.lru_cache(maxsize=None)
def _make_gather(bsz, seq, V, D, chunk):
    b_per_w = (bsz * seq) // _NW
    assert seq % b_per_w == 0
    w_per_row = seq // b_per_w
    n_chunks = b_per_w // chunk
    assert n_chunks % _NG == 0
    vregs_per_row = D // _LANES
    mesh = plsc.VectorSubcoreMesh(core_axis_name="c", subcore_axis_name="s")

    @functools.partial(
        pl.kernel,
        out_type=jax.ShapeDtypeStruct((bsz, seq, D), jnp.float32),
        mesh=mesh,
        scratch_types=(
            [pltpu.VMEM((b_per_w,), jnp.int32)]
            + [pltpu.VMEM((chunk, D), jnp.float32)] * (_NG + _NSB)
            + [pltpu.SemaphoreType.DMA] * (_NG + _NSB)
        ),
    )
    def k(tok_hbm, table_hbm, out_hbm, idx_v, *bufs_and_sems):
        g = bufs_and_sems[:_NG]
        s = bufs_and_sems[_NG:_NG + _NSB]
        gsem = bufs_and_sems[_NG + _NSB:2 * _NG + _NSB]
        ssem = bufs_and_sems[2 * _NG + _NSB:]

        wid = lax.axis_index("s") * _NC + lax.axis_index("c")
        bi = wid // w_per_row
        si = (wid % w_per_row) * b_per_w
        pltpu.sync_copy(tok_hbm.at[bi, pl.ds(si, b_per_w)], idx_v)

        def idx_slice(off):
            return idx_v.at[pl.ds(pl.multiple_of(off, 8), chunk)]

        def gather_wait(j):
            pltpu.make_async_copy(
                table_hbm.at[pl.ds(0, chunk)], g[j], gsem[j]).wait()

        def store_wait(j):
            pltpu.make_async_copy(
                s[j], out_hbm.at[0, pl.ds(0, chunk)], ssem[j]).wait()

        def scale_chunk(gbuf, sbuf):
            def row(r, _):
                for c in range(vregs_per_row):
                    sl = pl.ds(c * _LANES, _LANES)
                    sbuf[r, sl] = gbuf[r, sl] * SCALE
                return 0
            lax.fori_loop(0, chunk, row, 0)

        for j in range(_NG):
            pltpu.async_copy(table_hbm.at[idx_slice(j * chunk)], g[j], gsem[j])

        def body(i, _):
            for gj in range(_NG):
                q = _NG * i + gj
                sj = gj % _NSB
                off = pl.multiple_of(si + q * chunk, 8)
                gather_wait(gj)
                if gj >= _NSB:
                    store_wait(sj)
                else:
                    @pl.when(i > 0)
                    def _():
                        store_wait(sj)
                scale_chunk(g[gj], s[sj])
                pltpu.async_copy(
                    s[sj], out_hbm.at[bi, pl.ds(off, chunk)], ssem[sj])

                @pl.when(q + _NG < n_chunks)
                def _():
                    noff = pl.multiple_of((q + _NG) * chunk, 8)
                    pltpu.async_copy(
                        table_hbm.at[idx_slice(noff)], g[gj], gsem[gj])
            return 0

        lax.fori_loop(0, n_chunks // _NG, body, 0)
        for j in range(_NSB):
            store_wait(j)

    return k


def kernel(token, embedding):
    b, s = token.shape
    V, D = embedding.shape
    return _make_gather(b, s, V, D, 8)(token.astype(jnp.int32), embedding)


# speedup vs baseline: 1.0571x; 1.0571x over previous
import functools
import math

import jax
import jax.numpy as jnp
from jax import lax
from jax.experimental import pallas as pl
from jax.experimental.pallas import tpu as pltpu
from jax.experimental.pallas import tpu_sc as plsc

D_MODEL = 1024
SCALE = math.sqrt(D_MODEL)

_info = plsc.get_sparse_core_info()
_NC = _info.num_cores
_NS = _info.num_subcores
_LANES = _info.num_lanes
_NW = _NC * _NS

_NG = 4
_NSB = 2


@functools.lru_cache(maxsize=None)
def _make_gather(bsz, seq, V, D, chunk):
    b_per_w = (bsz * seq) // _NW
    assert seq % b_per_w == 0
    w_per_row = seq // b_per_w
    n_chunks = b_per_w // chunk
    assert n_chunks % _NG == 0
    vregs_per_row = D // _LANES
    mesh = plsc.VectorSubcoreMesh(core_axis_name="c", subcore_axis_name="s")

    @functools.partial(
        pl.kernel,
        out_type=jax.ShapeDtypeStruct((bsz, seq, D), jnp.float32),
        mesh=mesh,
        scratch_types=(
            [pltpu.VMEM((b_per_w,), jnp.int32)]
            + [pltpu.VMEM((chunk, D), jnp.float32)] * (_NG + _NSB)
            + [pltpu.SemaphoreType.DMA] * (_NG + _NSB)
        ),
    )
    def k(tok_hbm, table_hbm, out_hbm, idx_v, *bufs_and_sems):
        g = bufs_and_sems[:_NG]
        s = bufs_and_sems[_NG:_NG + _NSB]
        gsem = bufs_and_sems[_NG + _NSB:2 * _NG + _NSB]
        ssem = bufs_and_sems[2 * _NG + _NSB:]

        wid = lax.axis_index("s") * _NC + lax.axis_index("c")
        bi = wid // w_per_row
        si = (wid % w_per_row) * b_per_w
        pltpu.sync_copy(tok_hbm.at[bi, pl.ds(si, b_per_w)], idx_v)

        def idx_slice(off):
            return idx_v.at[pl.ds(pl.multiple_of(off, 8), chunk)]

        def gather_wait(j):
            pltpu.make_async_copy(
                table_hbm.at[pl.ds(0, chunk)], g[j], gsem[j]).wait()

        def store_wait(j):
            pltpu.make_async_copy(
                s[j], out_hbm.at[0, pl.ds(0, chunk)], ssem[j]).wait()

        def scale_chunk(gbuf, sbuf):
            def step(t, _):
                r = lax.div(t, 4)
                col0 = lax.rem(t, 4) * (16 * _LANES)
                for c in range(16):
                    sl = pl.ds(col0 + c * _LANES, _LANES)
                    sbuf[r, sl] = gbuf[r, sl] * SCALE
                return 0
            lax.fori_loop(0, chunk * 4, step, 0)

        for j in range(_NG):
            pltpu.async_copy(table_hbm.at[idx_slice(j * chunk)], g[j], gsem[j])

        def body(i, _):
            for gj in range(_NG):
                q = _NG * i + gj
                sj = gj % _NSB
                off = pl.multiple_of(si + q * chunk, 8)
                gather_wait(gj)
                if gj >= _NSB:
                    store_wait(sj)
                else:
                    @pl.when(i > 0)
                    def _():
                        store_wait(sj)
                scale_chunk(g[gj], s[sj])
                pltpu.async_copy(
                    s[sj], out_hbm.at[bi, pl.ds(off, chunk)], ssem[sj])

                @pl.when(q + _NG < n_chunks)
                def _():
                    noff = pl.multiple_of((q + _NG) * chunk, 8)
                    pltpu.async_copy(
                        table_hbm.at[idx_slice(noff)], g[gj], gsem[gj])
            return 0

        lax.fori_loop(0, n_chunks // _NG, body, 0)
        for j in range(_NSB):
            store_wait(j)

    return k


def kernel(token, embedding):
    b, s = token.shape
    V, D = embedding.shape
    return _make_gather(b, s, V, D, 16)(token.astype(jnp.int32), embedding)
